# baseline (speedup 1.0000x reference)
import functools

import jax
import jax.numpy as jnp
from jax import lax
from jax.experimental import pallas as pl
from jax.experimental.pallas import tpu as pltpu
from jax.experimental.pallas import tpu_sc as plsc

R = 4
N_NODES = 10000
N_EDGES = 160000
D = 128
VL = 16

NC = 2
NS = 16
B_TOTAL = R * N_EDGES
E_PER_S = N_EDGES // NS
CHUNK = 40
N_CHUNKS = E_PER_S // CHUNK
NBUF = 5
N_BLOCKS = N_NODES // CHUNK
BIG_LOADERS = N_BLOCKS - 15 * NS
BLK_MAX = 16
NSTAGE = NBUF


def _sc_kernel(table, aidx, attn_b):
    mesh = plsc.VectorSubcoreMesh(core_axis_name="c", subcore_axis_name="s")

    @functools.partial(
        pl.kernel,
        mesh=mesh,
        out_type=jax.ShapeDtypeStruct((B_TOTAL, D), jnp.float32),
        scratch_types=(
            [pltpu.VMEM((E_PER_S,), jnp.int32),
             pltpu.VMEM((NBUF, CHUNK, D), jnp.float32),
             pltpu.VMEM((VL,), jnp.float32),
             pltpu.VMEM_SHARED((N_NODES, D), jnp.float32)]
            + [pltpu.SemaphoreType.DMA] * (2 * NBUF + 1)
        ),
    )
    def k(table_hbm, idx_hbm, attn_hbm, out_hbm, idx_v, rows_v,
          attn_v, shared, *sems):
        gsem = sems[:NBUF]
        ssem = sems[NBUF:2 * NBUF]
        isem = sems[2 * NBUF]
        c = lax.axis_index("c")
        s = lax.axis_index("s")
        idx_cp = pltpu.make_async_copy(
            idx_hbm.at[pl.ds(s * E_PER_S, E_PER_S)], idx_v, isem)
        idx_cp.start()

        gates = []
        for p in range(2):
            pltpu.sync_copy(attn_hbm.at[pl.ds((NC * c + p) * VL, VL)], attn_v)
            gates.append(1.0 / (1.0 + jnp.exp(-attn_v[...])))

        blk_start = jnp.where(
            s < BIG_LOADERS,
            s * BLK_MAX,
            BIG_LOADERS * BLK_MAX + (s - BIG_LOADERS) * (BLK_MAX - 1),
        )
        n_blk = jnp.where(s < BIG_LOADERS, BLK_MAX, BLK_MAX - 1)

        for p in range(2):
            r = NC * c + p
            gate = gates[p]

            tab_base = r * N_NODES

            def in_start(k_, buf):
                pltpu.make_async_copy(
                    table_hbm.at[pl.ds(tab_base + (blk_start + k_) * CHUNK,
                                       CHUNK)],
                    rows_v.at[buf], gsem[buf],
                ).start()

            def in_wait(buf):
                pltpu.make_async_copy(
                    table_hbm.at[pl.ds(0, CHUNK)], rows_v.at[buf], gsem[buf]
                ).wait()

            def out_start(k_, buf):
                pltpu.make_async_copy(
                    rows_v.at[buf],
                    shared.at[pl.ds((blk_start + k_) * CHUNK, CHUNK)],
                    ssem[buf],
                ).start()

            def out_wait(buf):
                pltpu.make_async_copy(
                    rows_v.at[buf], shared.at[pl.ds(0, CHUNK)], ssem[buf]
                ).wait()

            def scale(buf):
                @pl.loop(0, CHUNK, step=8)
                def _(i):
                    for ii in range(8):
                        for j in range(D // VL):
                            sl = pl.ds(j * VL, VL)
                            rows_v[buf, i + ii, sl] = (
                                rows_v[buf, i + ii, sl] * gate)

            for b in range(NSTAGE):
                in_start(b, b)
            for k_ in range(BLK_MAX):
                buf = k_ % NSTAGE

                @pl.when(k_ < n_blk)
                def _():
                    in_wait(buf)
                    scale(buf)
                    out_start(k_, buf)

                if k_ + NSTAGE < BLK_MAX:
                    @pl.when(k_ + NSTAGE < n_blk)
                    def _():
                        out_wait(buf)
                        in_start(k_ + NSTAGE, buf)
            for b in range(NSTAGE):
                out_wait(b)
            plsc.subcore_barrier()

            if p == 0:
                idx_cp.wait()
            out_base = r * N_EDGES + s * E_PER_S

            def g_start(cc, buf):
                pltpu.make_async_copy(
                    shared.at[idx_v.at[pl.ds(cc * CHUNK, CHUNK)]],
                    rows_v.at[buf], gsem[buf],
                ).start()

            def g_wait(buf):
                pltpu.make_async_copy(
                    shared.at[pl.ds(0, CHUNK)], rows_v.at[buf], gsem[buf]
                ).wait()

            def s_start(cc, buf):
                pltpu.make_async_copy(
                    rows_v.at[buf],
                    out_hbm.at[pl.ds(out_base + cc * CHUNK, CHUNK)],
                    ssem[buf],
                ).start()

            def s_wait(buf):
                pltpu.make_async_copy(
                    rows_v.at[buf], out_hbm.at[pl.ds(0, CHUNK)], ssem[buf]
                ).wait()

            for b in range(NBUF):
                g_start(b, b)

            @pl.loop(0, N_CHUNKS, step=NBUF)
            def _(cc):
                for b in range(NBUF):
                    g_wait(b)
                    s_start(cc + b, b)

                @pl.when(cc + NBUF < N_CHUNKS)
                def _():
                    for b in range(NBUF):
                        s_wait(b)
                        g_start(cc + NBUF + b, b)

            for b in range(NBUF):
                s_wait(b)
            plsc.subcore_barrier()

    return k(table, aidx, attn_b)


def kernel(rel_attn, per_rel_msgs, actor_idx):
    msgs2d = per_rel_msgs.reshape(R * N_NODES, D)
    aidx = actor_idx.astype(jnp.int32)
    attn_b = jnp.broadcast_to(
        rel_attn.reshape(R, 1), (R, VL)).reshape(R * VL)
    out_flat = _sc_kernel(msgs2d, aidx, attn_b)
    return out_flat.reshape(R, N_EDGES, D)


# speedup vs baseline: 1.0304x; 1.0304x over previous
import functools

import jax
import jax.numpy as jnp
from jax import lax
from jax.experimental import pallas as pl
from jax.experimental.pallas import tpu as pltpu
from jax.experimental.pallas import tpu_sc as plsc

R = 4
N_NODES = 10000
N_EDGES = 160000
D = 128
VL = 16

NC = 2
NS = 16
B_TOTAL = R * N_EDGES
E_PER_S = N_EDGES // NS
CHUNK = 40
N_CHUNKS = E_PER_S // CHUNK
NBUF = 5
N_BLOCKS = N_NODES // CHUNK
BIG_LOADERS = N_BLOCKS - 15 * NS
BLK_MAX = 16
NSTAGE = NBUF


def _sc_kernel(table, aidx, attn_b):
    mesh = plsc.VectorSubcoreMesh(core_axis_name="c", subcore_axis_name="s")

    @functools.partial(
        pl.kernel,
        mesh=mesh,
        out_type=jax.ShapeDtypeStruct((B_TOTAL, D), jnp.float32),
        scratch_types=(
            [pltpu.VMEM((E_PER_S,), jnp.int32),
             pltpu.VMEM((NBUF, CHUNK, D), jnp.float32),
             pltpu.VMEM((VL,), jnp.float32),
             pltpu.VMEM_SHARED((N_NODES, D), jnp.float32)]
            + [pltpu.SemaphoreType.DMA] * (2 * NBUF + 1)
        ),
    )
    def k(table_hbm, idx_hbm, attn_hbm, out_hbm, idx_v, rows_v,
          attn_v, shared, *sems):
        gsem = sems[:NBUF]
        ssem = sems[NBUF:2 * NBUF]
        isem = sems[2 * NBUF]
        c = lax.axis_index("c")
        s = lax.axis_index("s")
        idx_cp = pltpu.make_async_copy(
            idx_hbm.at[pl.ds(s * E_PER_S, E_PER_S)], idx_v, isem)
        idx_cp.start()

        gates = []
        for p in range(2):
            pltpu.sync_copy(attn_hbm.at[pl.ds((NC * c + p) * VL, VL)], attn_v)
            gates.append(1.0 / (1.0 + jnp.exp(-attn_v[...])))

        blk_start = jnp.where(
            s < BIG_LOADERS,
            s * BLK_MAX,
            BIG_LOADERS * BLK_MAX + (s - BIG_LOADERS) * (BLK_MAX - 1),
        )
        n_blk = jnp.where(s < BIG_LOADERS, BLK_MAX, BLK_MAX - 1)

        for p in range(2):
            r = NC * c + p
            gate = gates[p]

            tab_base = r * N_NODES

            def in_start(k_, buf):
                pltpu.make_async_copy(
                    table_hbm.at[pl.ds(tab_base + (blk_start + k_) * CHUNK,
                                       CHUNK)],
                    rows_v.at[buf], gsem[buf],
                ).start()

            def in_wait(buf):
                pltpu.make_async_copy(
                    table_hbm.at[pl.ds(0, CHUNK)], rows_v.at[buf], gsem[buf]
                ).wait()

            def out_start(k_, buf):
                pltpu.make_async_copy(
                    rows_v.at[buf],
                    shared.at[pl.ds((blk_start + k_) * CHUNK, CHUNK)],
                    ssem[buf],
                ).start()

            def out_wait(buf):
                pltpu.make_async_copy(
                    rows_v.at[buf], shared.at[pl.ds(0, CHUNK)], ssem[buf]
                ).wait()

            def scale(buf):
                @pl.loop(0, CHUNK, step=4)
                def _(i):
                    for ii in range(4):
                        for j in range(D // VL):
                            sl = pl.ds(j * VL, VL)
                            rows_v[buf, i + ii, sl] = (
                                rows_v[buf, i + ii, sl] * gate)

            for b in range(NSTAGE):
                in_start(b, b)
            for k_ in range(BLK_MAX):
                buf = k_ % NSTAGE

                @pl.when(k_ < n_blk)
                def _():
                    in_wait(buf)
                    scale(buf)
                    out_start(k_, buf)

                if k_ + NSTAGE < BLK_MAX:
                    @pl.when(k_ + NSTAGE < n_blk)
                    def _():
                        out_wait(buf)
                        in_start(k_ + NSTAGE, buf)
            for b in range(NSTAGE):
                out_wait(b)
            plsc.subcore_barrier()

            if p == 0:
                idx_cp.wait()
            out_base = r * N_EDGES + s * E_PER_S

            def g_start(cc, buf):
                pltpu.make_async_copy(
                    shared.at[idx_v.at[pl.ds(cc * CHUNK, CHUNK)]],
                    rows_v.at[buf], gsem[buf],
                ).start()

            def g_wait(buf):
                pltpu.make_async_copy(
                    shared.at[pl.ds(0, CHUNK)], rows_v.at[buf], gsem[buf]
                ).wait()

            def s_start(cc, buf):
                pltpu.make_async_copy(
                    rows_v.at[buf],
                    out_hbm.at[pl.ds(out_base + cc * CHUNK, CHUNK)],
                    ssem[buf],
                ).start()

            def s_wait(buf):
                pltpu.make_async_copy(
                    rows_v.at[buf], out_hbm.at[pl.ds(0, CHUNK)], ssem[buf]
                ).wait()

            for b in range(NBUF):
                g_start(b, b)

            @pl.loop(0, N_CHUNKS, step=NBUF)
            def _(cc):
                for b in range(NBUF):
                    g_wait(b)
                    s_start(cc + b, b)

                @pl.when(cc + NBUF < N_CHUNKS)
                def _():
                    for b in range(NBUF):
                        s_wait(b)
                        g_start(cc + NBUF + b, b)

            for b in range(NBUF):
                s_wait(b)
            plsc.subcore_barrier()

    return k(table, aidx, attn_b)


def kernel(rel_attn, per_rel_msgs, actor_idx):
    msgs2d = per_rel_msgs.reshape(R * N_NODES, D)
    aidx = actor_idx.astype(jnp.int32)
    attn_b = jnp.broadcast_to(
        rel_attn.reshape(R, 1), (R, VL)).reshape(R * VL)
    out_flat = _sc_kernel(msgs2d, aidx, attn_b)
    return out_flat.reshape(R, N_EDGES, D)
